# Initial kernel scaffold; baseline (speedup 1.0000x reference)
import jax
import jax.numpy as jnp
from jax.experimental import pallas as pl


def kernel(x, edge_index, batch_index, W0, b0, W1, b1, W2, b2, W3, b3, Wfc, bfc):
    raise NotImplementedError("write your pallas kernel here")


# speedup vs baseline: 12.0436x; 12.0436x over previous
import functools

import jax
import jax.numpy as jnp
from jax import lax
from jax.experimental import pallas as pl
from jax.experimental.pallas import tpu as pltpu
from jax.experimental.pallas import tpu_sc as plsc

N = 50000
NP = 50176
DIN = 12
D = 64
G = 256
HALF = 25088
ACC_ROWS = 26624
TRASH = 25088
ZROWS = ACC_ROWS // 16
VROWS = HALF // 16
E = 800000
EPT = 50176
E_PAD = EPT * 16
GE = 1024
GROUPS = EPT // GE
RB = 512
GRID = NP // RB

_mesh = plsc.VectorSubcoreMesh(core_axis_name="c", subcore_axis_name="s")
_sc_params = pltpu.CompilerParams(use_tc_tiling_on_sc=False)


def _remap(dstage, lstage, lo):
    hi = lo + HALF
    for kb in range(8):
        for kk in range(8):
            d16 = dstage[pl.ds(kb * 128 + kk * 16, 16)]
            inr = (d16 >= lo) & (d16 < hi)
            loc = jnp.where(inr, d16 - lo, TRASH + (d16 & 1023))
            lstage[kb, pl.ds(kk * 16, 16)] = loc


@functools.partial(
    pl.kernel,
    mesh=_mesh,
    out_type=jax.ShapeDtypeStruct((NP,), jnp.float32),
    scratch_types=[
        pltpu.VMEM((GE,), jnp.int32),
        pltpu.VMEM((8, 128), jnp.int32),
        pltpu.VMEM((128,), jnp.float32),
        pltpu.VMEM((VROWS,), jnp.float32),
        pltpu.VMEM_SHARED((ACC_ROWS,), jnp.float32),
    ],
    compiler_params=_sc_params,
)
def _deg(dst_hbm, deg_hbm, dstage, lstage, ones_v, obuf, dacc):
    c = lax.axis_index("c")
    s = lax.axis_index("s")
    lo = c * HALF
    base = s * EPT
    for kk in range(8):
        ones_v[pl.ds(kk * 16, 16)] = jnp.zeros((16,), jnp.float32)
    for kb in range(ZROWS // 128):
        pltpu.sync_copy(ones_v, dacc.at[pl.ds(s * ZROWS + kb * 128, 128)])
    for kk in range(8):
        ones_v[pl.ds(kk * 16, 16)] = jnp.full((16,), 1.0, jnp.float32)
    plsc.subcore_barrier()

    def group(gi, carry):
        off = base + gi * GE
        pltpu.sync_copy(dst_hbm.at[pl.ds(off, GE)], dstage)
        _remap(dstage, lstage, lo)
        for kb in range(8):
            pltpu.sync_copy(ones_v, dacc.at[lstage.at[kb]], add=True)
        return carry

    lax.fori_loop(0, GROUPS, group, 0)
    plsc.subcore_barrier()
    pltpu.sync_copy(dacc.at[pl.ds(s * VROWS, VROWS)], obuf)
    pltpu.sync_copy(obuf, deg_hbm.at[pl.ds(c * HALF + s * VROWS, VROWS)])


@functools.partial(
    pl.kernel,
    mesh=_mesh,
    out_type=jax.ShapeDtypeStruct((NP, D), jnp.float32),
    scratch_types=[
        pltpu.VMEM((GE,), jnp.int32),
        pltpu.VMEM((GE,), jnp.int32),
        pltpu.VMEM((8, 128), jnp.int32),
        pltpu.VMEM((128, D), jnp.float32),
        pltpu.VMEM_SHARED((ACC_ROWS, D), jnp.float32),
    ],
    compiler_params=_sc_params,
)
def _agg(src_hbm, dst_hbm, g_hbm, z2_hbm, out_hbm,
         sstage, dstage, lstage, rows, acc):
    c = lax.axis_index("c")
    s = lax.axis_index("s")
    lo = c * HALF
    base = s * EPT
    pltpu.sync_copy(z2_hbm, rows)
    for kb in range(ZROWS // 128):
        pltpu.sync_copy(rows, acc.at[pl.ds(s * ZROWS + kb * 128, 128)])
    plsc.subcore_barrier()

    def group(gi, carry):
        off = base + gi * GE
        pltpu.sync_copy(src_hbm.at[pl.ds(off, GE)], sstage)
        pltpu.sync_copy(dst_hbm.at[pl.ds(off, GE)], dstage)
        _remap(dstage, lstage, lo)
        for kb in range(8):
            pltpu.sync_copy(g_hbm.at[sstage.at[pl.ds(kb * 128, 128)]], rows)
            pltpu.sync_copy(rows, acc.at[lstage.at[kb]], add=True)
        return carry

    lax.fori_loop(0, GROUPS, group, 0)
    plsc.subcore_barrier()
    for kb in range(12):
        pltpu.sync_copy(acc.at[pl.ds(s * VROWS + kb * 128, 128)], rows)
        pltpu.sync_copy(
            rows, out_hbm.at[pl.ds(c * HALF + s * VROWS + kb * 128, 128)])
    pltpu.sync_copy(acc.at[pl.ds(s * VROWS + 1536, 32)],
                    rows.at[pl.ds(0, 32)])
    pltpu.sync_copy(rows.at[pl.ds(0, 32)],
                    out_hbm.at[pl.ds(c * HALF + s * VROWS + 1536, 32)])


def _g0_body(x_ref, w_ref, deg_ref, g_ref):
    dinv = lax.rsqrt(deg_ref[...] + 1.0)
    g_ref[...] = jnp.dot(x_ref[...], w_ref[...],
                         preferred_element_type=jnp.float32) * dinv


_g0 = pl.pallas_call(
    _g0_body,
    grid=(GRID,),
    in_specs=[
        pl.BlockSpec((RB, DIN), lambda i: (i, 0)),
        pl.BlockSpec((DIN, D), lambda i: (0, 0)),
        pl.BlockSpec((RB, 1), lambda i: (i, 0)),
    ],
    out_specs=pl.BlockSpec((RB, D), lambda i: (i, 0)),
    out_shape=jax.ShapeDtypeStruct((NP, D), jnp.float32),
)


def _mid_body(acc_ref, g_ref, deg_ref, b_ref, w_ref, out_ref):
    dinv = lax.rsqrt(deg_ref[...] + 1.0)
    pre = (acc_ref[...] + g_ref[...]) * dinv + b_ref[...]
    xl = jnp.maximum(pre, 0.01 * pre)
    out_ref[...] = jnp.dot(xl, w_ref[...],
                           preferred_element_type=jnp.float32) * dinv


_mid = pl.pallas_call(
    _mid_body,
    grid=(GRID,),
    in_specs=[
        pl.BlockSpec((RB, D), lambda i: (i, 0)),
        pl.BlockSpec((RB, D), lambda i: (i, 0)),
        pl.BlockSpec((RB, 1), lambda i: (i, 0)),
        pl.BlockSpec((1, D), lambda i: (0, 0)),
        pl.BlockSpec((D, D), lambda i: (0, 0)),
    ],
    out_specs=pl.BlockSpec((RB, D), lambda i: (i, 0)),
    out_shape=jax.ShapeDtypeStruct((NP, D), jnp.float32),
)


def _final_body(acc_ref, g_ref, deg_ref, b_ref, bi_ref, wfc_ref, bfc_ref,
                out_ref, sums, counts):
    i = pl.program_id(0)

    @pl.when(i == 0)
    def _():
        sums[...] = jnp.zeros_like(sums)
        counts[...] = jnp.zeros_like(counts)

    dinv = lax.rsqrt(deg_ref[...] + 1.0)
    h = jnp.maximum((acc_ref[...] + g_ref[...]) * dinv + b_ref[...], 0.0)
    onehot = (bi_ref[...] == lax.broadcasted_iota(jnp.int32, (RB, G), 1)
              ).astype(jnp.float32)
    dn = (((0,), (0,)), ((), ()))
    sums[...] += lax.dot_general(onehot, h, dn,
                                 preferred_element_type=jnp.float32)
    counts[...] += lax.dot_general(onehot, jnp.ones((RB, 1), jnp.float32), dn,
                                   preferred_element_type=jnp.float32)

    @pl.when(i == GRID - 1)
    def _():
        mean = sums[...] / jnp.maximum(counts[...], 1.0)
        z = jnp.dot(mean, wfc_ref[...],
                    preferred_element_type=jnp.float32) + bfc_ref[...]
        out_ref[...] = jax.nn.sigmoid(z)


_final = pl.pallas_call(
    _final_body,
    grid=(GRID,),
    in_specs=[
        pl.BlockSpec((RB, D), lambda i: (i, 0)),
        pl.BlockSpec((RB, D), lambda i: (i, 0)),
        pl.BlockSpec((RB, 1), lambda i: (i, 0)),
        pl.BlockSpec((1, D), lambda i: (0, 0)),
        pl.BlockSpec((RB, 1), lambda i: (i, 0)),
        pl.BlockSpec((D, 1), lambda i: (0, 0)),
        pl.BlockSpec((1, 1), lambda i: (0, 0)),
    ],
    out_specs=pl.BlockSpec((G, 1), lambda i: (0, 0)),
    out_shape=jax.ShapeDtypeStruct((G, 1), jnp.float32),
    scratch_shapes=[
        pltpu.VMEM((G, D), jnp.float32),
        pltpu.VMEM((G, 1), jnp.float32),
    ],
)


def kernel(x, edge_index, batch_index, W0, b0, W1, b1, W2, b2, W3, b3,
           Wfc, bfc):
    src = edge_index[0].astype(jnp.int32)
    dst = edge_index[1].astype(jnp.int32)
    pad_e = E_PAD - E
    src_p = jnp.concatenate(
        [src, (jnp.arange(pad_e, dtype=jnp.int32) % 64)])
    dst_p = jnp.concatenate(
        [dst, jnp.full((pad_e,), 1 << 29, jnp.int32)])
    xp = jnp.zeros((NP, DIN), jnp.float32).at[:N].set(x)
    bi = jnp.concatenate(
        [batch_index.astype(jnp.int32),
         jnp.full((NP - N,), G + 7, jnp.int32)]).reshape(NP, 1)
    zeros2 = jnp.zeros((128, D), jnp.float32)

    deg = _deg(dst_p)
    deg2 = deg.reshape(NP, 1)
    g = _g0(xp, W0, deg2)
    acc = _agg(src_p, dst_p, g, zeros2)
    for bprev, wnext in ((b0, W1), (b1, W2), (b2, W3)):
        g = _mid(acc, g, deg2, bprev.reshape(1, D), wnext)
        acc = _agg(src_p, dst_p, g, zeros2)
    return _final(acc, g, deg2, b3.reshape(1, D), bi,
                  Wfc, bfc.reshape(1, 1))


# speedup vs baseline: 19.7515x; 1.6400x over previous
import functools

import jax
import jax.numpy as jnp
from jax import lax
from jax.experimental import pallas as pl
from jax.experimental.pallas import tpu as pltpu
from jax.experimental.pallas import tpu_sc as plsc

N = 50000
NP = 50176
DIN = 12
D = 64
G = 256
HALF = 25088
ACC_ROWS = 26624
TRASH = 25088
ZROWS = ACC_ROWS // 16
VROWS = HALF // 16
E = 800000
EPT = 50176
E_PAD = EPT * 16
GE = 1024
GROUPS = EPT // GE
AGE = 512
AGROUPS = EPT // AGE
CPG = AGE // 128
RB = 512
GRID = NP // RB

_mesh = plsc.VectorSubcoreMesh(core_axis_name="c", subcore_axis_name="s")
_sc_params = pltpu.CompilerParams(use_tc_tiling_on_sc=False)


def _remap(dstage, lstage, lo):
    hi = lo + HALF
    for kb in range(8):
        for kk in range(8):
            d16 = dstage[pl.ds(kb * 128 + kk * 16, 16)]
            inr = (d16 >= lo) & (d16 < hi)
            loc = jnp.where(inr, d16 - lo, TRASH + (d16 & 1023))
            lstage[kb, pl.ds(kk * 16, 16)] = loc


@functools.partial(
    pl.kernel,
    mesh=_mesh,
    out_type=jax.ShapeDtypeStruct((NP,), jnp.float32),
    scratch_types=[
        pltpu.VMEM((GE,), jnp.int32),
        pltpu.VMEM((8, 128), jnp.int32),
        pltpu.VMEM((128,), jnp.float32),
        pltpu.VMEM((VROWS,), jnp.float32),
        pltpu.VMEM_SHARED((ACC_ROWS,), jnp.float32),
    ],
    compiler_params=_sc_params,
)
def _deg(dst_hbm, deg_hbm, dstage, lstage, ones_v, obuf, dacc):
    c = lax.axis_index("c")
    s = lax.axis_index("s")
    lo = c * HALF
    base = s * EPT
    for kk in range(8):
        ones_v[pl.ds(kk * 16, 16)] = jnp.zeros((16,), jnp.float32)
    for kb in range(ZROWS // 128):
        pltpu.sync_copy(ones_v, dacc.at[pl.ds(s * ZROWS + kb * 128, 128)])
    for kk in range(8):
        ones_v[pl.ds(kk * 16, 16)] = jnp.full((16,), 1.0, jnp.float32)
    plsc.subcore_barrier()

    def group(gi, carry):
        off = base + gi * GE
        pltpu.sync_copy(dst_hbm.at[pl.ds(off, GE)], dstage)
        _remap(dstage, lstage, lo)
        for kb in range(8):
            pltpu.sync_copy(ones_v, dacc.at[lstage.at[kb]], add=True)
        return carry

    lax.fori_loop(0, GROUPS, group, 0)
    plsc.subcore_barrier()
    pltpu.sync_copy(dacc.at[pl.ds(s * VROWS, VROWS)], obuf)
    pltpu.sync_copy(obuf, deg_hbm.at[pl.ds(c * HALF + s * VROWS, VROWS)])


@functools.partial(
    pl.kernel,
    mesh=_mesh,
    out_type=jax.ShapeDtypeStruct((NP, D), jnp.float32),
    scratch_types=[
        pltpu.VMEM((2, AGE), jnp.int32),
        pltpu.VMEM((2, AGE), jnp.int32),
        pltpu.VMEM((2, CPG, 128), jnp.int32),
        pltpu.VMEM((2, 128, D), jnp.float32),
        pltpu.VMEM_SHARED((ACC_ROWS, D), jnp.float32),
        pltpu.SemaphoreType.DMA((2, 2)),
        pltpu.SemaphoreType.DMA((2,)),
        pltpu.SemaphoreType.DMA((2,)),
    ],
    compiler_params=_sc_params,
)
def _agg(src_hbm, dst_hbm, g_hbm, z2_hbm, out_hbm,
         sstage, dstage, lstage, rows, acc, sem_st, sem_g, sem_s):
    c = lax.axis_index("c")
    s = lax.axis_index("s")
    lo = c * HALF
    base = s * EPT
    pltpu.sync_copy(z2_hbm, rows.at[0])
    for kb in range(ZROWS // 128):
        pltpu.sync_copy(rows.at[0], acc.at[pl.ds(s * ZROWS + kb * 128, 128)])
    plsc.subcore_barrier()

    def stage(gidx, p):
        off = base + gidx * AGE
        pltpu.async_copy(src_hbm.at[pl.ds(off, AGE)], sstage.at[p],
                         sem_st.at[p, 0])
        pltpu.async_copy(dst_hbm.at[pl.ds(off, AGE)], dstage.at[p],
                         sem_st.at[p, 1])

    def wait_stage(p):
        pltpu.make_async_copy(src_hbm.at[pl.ds(0, AGE)], sstage.at[p],
                              sem_st.at[p, 0]).wait()
        pltpu.make_async_copy(dst_hbm.at[pl.ds(0, AGE)], dstage.at[p],
                              sem_st.at[p, 1]).wait()

    def gath(p, b, slot):
        pltpu.async_copy(g_hbm.at[sstage.at[p, pl.ds(b * 128, 128)]],
                         rows.at[slot], sem_g.at[slot])

    def scat(p, b, slot):
        pltpu.async_copy(rows.at[slot], acc.at[lstage.at[p, b]],
                         sem_s.at[slot], add=True)

    def wait_scat(p, b, slot):
        pltpu.make_async_copy(rows.at[slot], acc.at[lstage.at[p, b]],
                              sem_s.at[slot]).wait()

    def wait_gath(p, b, slot):
        pltpu.make_async_copy(g_hbm.at[sstage.at[p, pl.ds(b * 128, 128)]],
                              rows.at[slot], sem_g.at[slot]).wait()

    def remap(p):
        for kb in range(CPG):
            for kk in range(8):
                d16 = dstage[p, pl.ds(kb * 128 + kk * 16, 16)]
                inr = (d16 >= lo) & (d16 < lo + HALF)
                loc = jnp.where(inr, d16 - lo, TRASH + (d16 & 1023))
                lstage[p, kb, pl.ds(kk * 16, 16)] = loc

    stage(0, 0)

    def pair(k, carry):
        for idx in range(8):
            p, b, slot = idx // 4, idx % 4, idx % 2
            if idx == 0:
                wait_stage(0)
                remap(0)
            if idx == 4:
                wait_stage(1)
                remap(1)
            if idx < 2:
                @pl.when(k > 0)
                def _():
                    wait_scat(1, 2 + idx, idx)
            else:
                wait_scat((idx - 2) // 4, (idx - 2) % 4, slot)
            gath(p, b, slot)
            if idx == 0:
                @pl.when(k > 0)
                def _():
                    wait_gath(1, 3, 1)
                    scat(1, 3, 1)
                stage(2 * k + 1, 1)
            else:
                pi, bi = (idx - 1) // 4, (idx - 1) % 4
                wait_gath(pi, bi, 1 - slot)
                scat(pi, bi, 1 - slot)
                if idx == 4:
                    @pl.when(k + 1 < AGROUPS // 2)
                    def _():
                        stage(2 * k + 2, 0)
        return carry

    lax.fori_loop(0, AGROUPS // 2, pair, 0)
    wait_gath(1, 3, 1)
    scat(1, 3, 1)
    wait_scat(1, 2, 0)
    wait_scat(1, 3, 1)
    plsc.subcore_barrier()
    for kb in range(12):
        pltpu.sync_copy(acc.at[pl.ds(s * VROWS + kb * 128, 128)],
                        rows.at[0])
        pltpu.sync_copy(
            rows.at[0],
            out_hbm.at[pl.ds(c * HALF + s * VROWS + kb * 128, 128)])
    pltpu.sync_copy(acc.at[pl.ds(s * VROWS + 1536, 32)],
                    rows.at[0, pl.ds(0, 32)])
    pltpu.sync_copy(rows.at[0, pl.ds(0, 32)],
                    out_hbm.at[pl.ds(c * HALF + s * VROWS + 1536, 32)])


def _g0_body(x_ref, w_ref, deg_ref, g_ref):
    dinv = lax.rsqrt(deg_ref[...] + 1.0)
    g_ref[...] = jnp.dot(x_ref[...], w_ref[...],
                         preferred_element_type=jnp.float32) * dinv


_g0 = pl.pallas_call(
    _g0_body,
    grid=(GRID,),
    in_specs=[
        pl.BlockSpec((RB, DIN), lambda i: (i, 0)),
        pl.BlockSpec((DIN, D), lambda i: (0, 0)),
        pl.BlockSpec((RB, 1), lambda i: (i, 0)),
    ],
    out_specs=pl.BlockSpec((RB, D), lambda i: (i, 0)),
    out_shape=jax.ShapeDtypeStruct((NP, D), jnp.float32),
)


def _mid_body(acc_ref, g_ref, deg_ref, b_ref, w_ref, out_ref):
    dinv = lax.rsqrt(deg_ref[...] + 1.0)
    pre = (acc_ref[...] + g_ref[...]) * dinv + b_ref[...]
    xl = jnp.maximum(pre, 0.01 * pre)
    out_ref[...] = jnp.dot(xl, w_ref[...],
                           preferred_element_type=jnp.float32) * dinv


_mid = pl.pallas_call(
    _mid_body,
    grid=(GRID,),
    in_specs=[
        pl.BlockSpec((RB, D), lambda i: (i, 0)),
        pl.BlockSpec((RB, D), lambda i: (i, 0)),
        pl.BlockSpec((RB, 1), lambda i: (i, 0)),
        pl.BlockSpec((1, D), lambda i: (0, 0)),
        pl.BlockSpec((D, D), lambda i: (0, 0)),
    ],
    out_specs=pl.BlockSpec((RB, D), lambda i: (i, 0)),
    out_shape=jax.ShapeDtypeStruct((NP, D), jnp.float32),
)


def _final_body(acc_ref, g_ref, deg_ref, b_ref, bi_ref, wfc_ref, bfc_ref,
                out_ref, sums, counts):
    i = pl.program_id(0)

    @pl.when(i == 0)
    def _():
        sums[...] = jnp.zeros_like(sums)
        counts[...] = jnp.zeros_like(counts)

    dinv = lax.rsqrt(deg_ref[...] + 1.0)
    h = jnp.maximum((acc_ref[...] + g_ref[...]) * dinv + b_ref[...], 0.0)
    onehot = (bi_ref[...] == lax.broadcasted_iota(jnp.int32, (RB, G), 1)
              ).astype(jnp.float32)
    dn = (((0,), (0,)), ((), ()))
    sums[...] += lax.dot_general(onehot, h, dn,
                                 preferred_element_type=jnp.float32)
    counts[...] += lax.dot_general(onehot, jnp.ones((RB, 1), jnp.float32), dn,
                                   preferred_element_type=jnp.float32)

    @pl.when(i == GRID - 1)
    def _():
        mean = sums[...] / jnp.maximum(counts[...], 1.0)
        z = jnp.dot(mean, wfc_ref[...],
                    preferred_element_type=jnp.float32) + bfc_ref[...]
        out_ref[...] = jax.nn.sigmoid(z)


_final = pl.pallas_call(
    _final_body,
    grid=(GRID,),
    in_specs=[
        pl.BlockSpec((RB, D), lambda i: (i, 0)),
        pl.BlockSpec((RB, D), lambda i: (i, 0)),
        pl.BlockSpec((RB, 1), lambda i: (i, 0)),
        pl.BlockSpec((1, D), lambda i: (0, 0)),
        pl.BlockSpec((RB, 1), lambda i: (i, 0)),
        pl.BlockSpec((D, 1), lambda i: (0, 0)),
        pl.BlockSpec((1, 1), lambda i: (0, 0)),
    ],
    out_specs=pl.BlockSpec((G, 1), lambda i: (0, 0)),
    out_shape=jax.ShapeDtypeStruct((G, 1), jnp.float32),
    scratch_shapes=[
        pltpu.VMEM((G, D), jnp.float32),
        pltpu.VMEM((G, 1), jnp.float32),
    ],
)


def kernel(x, edge_index, batch_index, W0, b0, W1, b1, W2, b2, W3, b3,
           Wfc, bfc):
    src = edge_index[0].astype(jnp.int32)
    dst = edge_index[1].astype(jnp.int32)
    pad_e = E_PAD - E
    src_p = jnp.concatenate(
        [src, (jnp.arange(pad_e, dtype=jnp.int32) % 64)])
    dst_p = jnp.concatenate(
        [dst, jnp.full((pad_e,), 1 << 29, jnp.int32)])
    xp = jnp.zeros((NP, DIN), jnp.float32).at[:N].set(x)
    bi = jnp.concatenate(
        [batch_index.astype(jnp.int32),
         jnp.full((NP - N,), G + 7, jnp.int32)]).reshape(NP, 1)
    zeros2 = jnp.zeros((128, D), jnp.float32)

    deg = _deg(dst_p)
    deg2 = deg.reshape(NP, 1)
    g = _g0(xp, W0, deg2)
    acc = _agg(src_p, dst_p, g, zeros2)
    for bprev, wnext in ((b0, W1), (b1, W2), (b2, W3)):
        g = _mid(acc, g, deg2, bprev.reshape(1, D), wnext)
        acc = _agg(src_p, dst_p, g, zeros2)
    return _final(acc, g, deg2, b3.reshape(1, D), bi,
                  Wfc, bfc.reshape(1, 1))


# speedup vs baseline: 27.3484x; 1.3846x over previous
import functools

import jax
import jax.numpy as jnp
from jax import lax
from jax.experimental import pallas as pl
from jax.experimental.pallas import tpu as pltpu
from jax.experimental.pallas import tpu_sc as plsc

N = 50000
NP = 50176
DIN = 12
D = 64
G = 256
HALF = 25088
ACC_ROWS = 26624
TRASH = 25088
ZROWS = ACC_ROWS // 16
VROWS = HALF // 16
E = 800000
EPT = 50176
E_PAD = EPT * 16
GE = 512
GROUPS = EPT // GE
AGE = 512
CPG = AGE // 128
CAP = 51200
RB = 512
GRID = NP // RB

_mesh = plsc.VectorSubcoreMesh(core_axis_name="c", subcore_axis_name="s")
_sc_params = pltpu.CompilerParams(use_tc_tiling_on_sc=False,
                                  needs_layout_passes=False)


@functools.partial(
    pl.kernel,
    mesh=_mesh,
    out_type=[
        jax.ShapeDtypeStruct((NP,), jnp.float32),
        jax.ShapeDtypeStruct((32, CAP), jnp.int32),
        jax.ShapeDtypeStruct((32, 16), jnp.int32),
    ],
    scratch_types=[
        pltpu.VMEM((2, GE), jnp.int32),
        pltpu.VMEM((2, GE), jnp.int32),
        pltpu.VMEM((2, CPG, 128), jnp.int32),
        pltpu.VMEM((CAP,), jnp.int32),
        pltpu.VMEM((128,), jnp.float32),
        pltpu.VMEM((16,), jnp.int32),
        pltpu.VMEM((VROWS,), jnp.float32),
        pltpu.VMEM_SHARED((ACC_ROWS,), jnp.float32),
        pltpu.SemaphoreType.DMA((2, 2)),
        pltpu.SemaphoreType.DMA((2, CPG)),
    ],
    compiler_params=_sc_params,
)
def _part(src_hbm, dst_hbm, deg_hbm, pl_hbm, cnt_hbm,
          sstage, dstage, lstage, pbuf, ones_v, cbuf, obuf, dacc,
          sem_st, sem_d):
    c = lax.axis_index("c")
    s = lax.axis_index("s")
    wid = c * 16 + s
    lo = c * HALF
    base = s * EPT
    for kk in range(8):
        ones_v[pl.ds(kk * 16, 16)] = jnp.zeros((16,), jnp.float32)
    for kb in range(ZROWS // 128):
        pltpu.sync_copy(ones_v, dacc.at[pl.ds(s * ZROWS + kb * 128, 128)])
    for kk in range(8):
        ones_v[pl.ds(kk * 16, 16)] = jnp.full((16,), 1.0, jnp.float32)
    plsc.subcore_barrier()

    def stage(gidx, p):
        off = base + gidx * GE
        pltpu.async_copy(src_hbm.at[pl.ds(off, GE)], sstage.at[p],
                         sem_st.at[p, 0])
        pltpu.async_copy(dst_hbm.at[pl.ds(off, GE)], dstage.at[p],
                         sem_st.at[p, 1])

    def wait_stage(p):
        pltpu.make_async_copy(src_hbm.at[pl.ds(0, GE)], sstage.at[p],
                              sem_st.at[p, 0]).wait()
        pltpu.make_async_copy(dst_hbm.at[pl.ds(0, GE)], dstage.at[p],
                              sem_st.at[p, 1]).wait()

    stage(0, 0)
    stage(1, 1)

    def pair(k, cnt):
        for p in range(2):
            gidx = 2 * k + p
            wait_stage(p)
            for kb in range(CPG):
                @pl.when(k > 0)
                def _():
                    pltpu.make_async_copy(ones_v, dacc.at[lstage.at[p, kb]],
                                          sem_d.at[p, kb]).wait()
                for kk in range(8):
                    i0 = kb * 128 + kk * 16
                    d16 = dstage[p, pl.ds(i0, 16)]
                    s16 = sstage[p, pl.ds(i0, 16)]
                    inr = (d16 >= lo) & (d16 < lo + HALF)
                    loc = jnp.where(inr, d16 - lo, TRASH + (d16 & 1023))
                    lstage[p, kb, pl.ds(kk * 16, 16)] = loc
                    key = jnp.where(inr, 0, 1).astype(jnp.int32)
                    pk = s16 | (loc << 16)
                    _, vs = plsc.sort_key_val(key, pk)
                    pbuf[pl.ds(cnt, 16)] = vs
                    cnt = cnt + jnp.sum(inr.astype(jnp.int32))
                pltpu.async_copy(ones_v, dacc.at[lstage.at[p, kb]],
                                 sem_d.at[p, kb], add=True)

            @pl.when(gidx + 2 < GROUPS)
            def _():
                stage(gidx + 2, p)
        return cnt

    cnt = lax.fori_loop(0, GROUPS // 2, pair, jnp.int32(0))
    for p in range(2):
        for kb in range(CPG):
            pltpu.make_async_copy(ones_v, dacc.at[lstage.at[p, kb]],
                                  sem_d.at[p, kb]).wait()
    lane = lax.iota(jnp.int32, 16)
    for kk in range(64):
        idx = lane + kk * 16
        pbuf[pl.ds(cnt + kk * 16, 16)] = (idx % 64) | ((TRASH + (idx & 1023)) << 16)
    cnt_pad = ((cnt + 1023) // 1024) * 1024
    cbuf[pl.ds(0, 16)] = jnp.zeros((16,), jnp.int32) + cnt_pad
    pltpu.sync_copy(cbuf, cnt_hbm.at[wid])

    def flush(j, carry):
        pltpu.sync_copy(pbuf.at[pl.ds(j * 2048, 2048)],
                        pl_hbm.at[wid, pl.ds(j * 2048, 2048)])
        return carry

    lax.fori_loop(0, (cnt_pad + 2047) // 2048, flush, 0)
    plsc.subcore_barrier()
    pltpu.sync_copy(dacc.at[pl.ds(s * VROWS, VROWS)], obuf)
    pltpu.sync_copy(obuf, deg_hbm.at[pl.ds(c * HALF + s * VROWS, VROWS)])


@functools.partial(
    pl.kernel,
    mesh=_mesh,
    out_type=jax.ShapeDtypeStruct((NP, D), jnp.float32),
    scratch_types=[
        pltpu.VMEM((2, AGE), jnp.int32),
        pltpu.VMEM((2, AGE), jnp.int32),
        pltpu.VMEM((2, CPG, 128), jnp.int32),
        pltpu.VMEM((2, 128, D), jnp.float32),
        pltpu.VMEM((16,), jnp.int32),
        pltpu.VMEM_SHARED((ACC_ROWS, D), jnp.float32),
        pltpu.SemaphoreType.DMA((2, 2)),
        pltpu.SemaphoreType.DMA((2,)),
        pltpu.SemaphoreType.DMA((2,)),
    ],
    compiler_params=_sc_params,
)
def _agg(pl_hbm, cnt_hbm, g_hbm, z2_hbm, out_hbm,
         pstage, sstage, lstage, rows, cbuf, acc, sem_st, sem_g, sem_s):
    c = lax.axis_index("c")
    s = lax.axis_index("s")
    wid = c * 16 + s
    pltpu.sync_copy(cnt_hbm.at[wid], cbuf)
    npairs = jnp.max(cbuf[...]) // 1024
    pltpu.sync_copy(z2_hbm, rows.at[0])
    for kb in range(ZROWS // 128):
        pltpu.sync_copy(rows.at[0], acc.at[pl.ds(s * ZROWS + kb * 128, 128)])
    plsc.subcore_barrier()

    def stage(gidx, p):
        pltpu.async_copy(pl_hbm.at[wid, pl.ds(gidx * AGE, AGE)],
                         pstage.at[p], sem_st.at[p, 0])

    def wait_stage(p):
        pltpu.make_async_copy(pl_hbm.at[wid, pl.ds(0, AGE)], pstage.at[p],
                              sem_st.at[p, 0]).wait()

    def unpack(p):
        for kb in range(CPG):
            for kk in range(8):
                v16 = pstage[p, pl.ds(kb * 128 + kk * 16, 16)]
                sstage[p, pl.ds(kb * 128 + kk * 16, 16)] = v16 & 0xFFFF
                lstage[p, kb, pl.ds(kk * 16, 16)] = (
                    lax.shift_right_logical(v16, 16))

    def gath(p, b, slot):
        pltpu.async_copy(g_hbm.at[sstage.at[p, pl.ds(b * 128, 128)]],
                         rows.at[slot], sem_g.at[slot])

    def scat(p, b, slot):
        pltpu.async_copy(rows.at[slot], acc.at[lstage.at[p, b]],
                         sem_s.at[slot], add=True)

    def wait_scat(p, b, slot):
        pltpu.make_async_copy(rows.at[slot], acc.at[lstage.at[p, b]],
                              sem_s.at[slot]).wait()

    def wait_gath(p, b, slot):
        pltpu.make_async_copy(g_hbm.at[sstage.at[p, pl.ds(b * 128, 128)]],
                              rows.at[slot], sem_g.at[slot]).wait()

    @pl.when(npairs > 0)
    def _():
        stage(0, 0)

    def pair(k, carry):
        for idx in range(8):
            p, b, slot = idx // 4, idx % 4, idx % 2
            if idx == 0:
                wait_stage(0)
                unpack(0)
            if idx == 4:
                wait_stage(1)
                unpack(1)
            if idx < 2:
                @pl.when(k > 0)
                def _():
                    wait_scat(1, 2 + idx, idx)
            else:
                wait_scat((idx - 2) // 4, (idx - 2) % 4, slot)
            gath(p, b, slot)
            if idx == 0:
                @pl.when(k > 0)
                def _():
                    wait_gath(1, 3, 1)
                    scat(1, 3, 1)
                stage(2 * k + 1, 1)
            else:
                pi, bi = (idx - 1) // 4, (idx - 1) % 4
                wait_gath(pi, bi, 1 - slot)
                scat(pi, bi, 1 - slot)
                if idx == 4:
                    @pl.when(k + 1 < npairs)
                    def _():
                        stage(2 * k + 2, 0)
        return carry

    lax.fori_loop(0, npairs, pair, 0)

    @pl.when(npairs > 0)
    def _():
        wait_gath(1, 3, 1)
        scat(1, 3, 1)
        wait_scat(1, 2, 0)
        wait_scat(1, 3, 1)

    plsc.subcore_barrier()
    for kb in range(12):
        pltpu.sync_copy(acc.at[pl.ds(s * VROWS + kb * 128, 128)],
                        rows.at[0])
        pltpu.sync_copy(
            rows.at[0],
            out_hbm.at[pl.ds(c * HALF + s * VROWS + kb * 128, 128)])
    pltpu.sync_copy(acc.at[pl.ds(s * VROWS + 1536, 32)],
                    rows.at[0, pl.ds(0, 32)])
    pltpu.sync_copy(rows.at[0, pl.ds(0, 32)],
                    out_hbm.at[pl.ds(c * HALF + s * VROWS + 1536, 32)])


def _g0_body(x_ref, w_ref, deg_ref, g_ref):
    dinv = lax.rsqrt(deg_ref[...] + 1.0)
    g_ref[...] = jnp.dot(x_ref[...], w_ref[...],
                         preferred_element_type=jnp.float32) * dinv


_g0 = pl.pallas_call(
    _g0_body,
    grid=(GRID,),
    in_specs=[
        pl.BlockSpec((RB, DIN), lambda i: (i, 0)),
        pl.BlockSpec((DIN, D), lambda i: (0, 0)),
        pl.BlockSpec((RB, 1), lambda i: (i, 0)),
    ],
    out_specs=pl.BlockSpec((RB, D), lambda i: (i, 0)),
    out_shape=jax.ShapeDtypeStruct((NP, D), jnp.float32),
)


def _mid_body(acc_ref, g_ref, deg_ref, b_ref, w_ref, out_ref):
    dinv = lax.rsqrt(deg_ref[...] + 1.0)
    pre = (acc_ref[...] + g_ref[...]) * dinv + b_ref[...]
    xl = jnp.maximum(pre, 0.01 * pre)
    out_ref[...] = jnp.dot(xl, w_ref[...],
                           preferred_element_type=jnp.float32) * dinv


_mid = pl.pallas_call(
    _mid_body,
    grid=(GRID,),
    in_specs=[
        pl.BlockSpec((RB, D), lambda i: (i, 0)),
        pl.BlockSpec((RB, D), lambda i: (i, 0)),
        pl.BlockSpec((RB, 1), lambda i: (i, 0)),
        pl.BlockSpec((1, D), lambda i: (0, 0)),
        pl.BlockSpec((D, D), lambda i: (0, 0)),
    ],
    out_specs=pl.BlockSpec((RB, D), lambda i: (i, 0)),
    out_shape=jax.ShapeDtypeStruct((NP, D), jnp.float32),
)


def _final_body(acc_ref, g_ref, deg_ref, b_ref, bi_ref, wfc_ref, bfc_ref,
                out_ref, sums, counts):
    i = pl.program_id(0)

    @pl.when(i == 0)
    def _():
        sums[...] = jnp.zeros_like(sums)
        counts[...] = jnp.zeros_like(counts)

    dinv = lax.rsqrt(deg_ref[...] + 1.0)
    h = jnp.maximum((acc_ref[...] + g_ref[...]) * dinv + b_ref[...], 0.0)
    onehot = (bi_ref[...] == lax.broadcasted_iota(jnp.int32, (RB, G), 1)
              ).astype(jnp.float32)
    dn = (((0,), (0,)), ((), ()))
    sums[...] += lax.dot_general(onehot, h, dn,
                                 preferred_element_type=jnp.float32)
    counts[...] += lax.dot_general(onehot, jnp.ones((RB, 1), jnp.float32), dn,
                                   preferred_element_type=jnp.float32)

    @pl.when(i == GRID - 1)
    def _():
        mean = sums[...] / jnp.maximum(counts[...], 1.0)
        z = jnp.dot(mean, wfc_ref[...],
                    preferred_element_type=jnp.float32) + bfc_ref[...]
        out_ref[...] = jax.nn.sigmoid(z)


_final = pl.pallas_call(
    _final_body,
    grid=(GRID,),
    in_specs=[
        pl.BlockSpec((RB, D), lambda i: (i, 0)),
        pl.BlockSpec((RB, D), lambda i: (i, 0)),
        pl.BlockSpec((RB, 1), lambda i: (i, 0)),
        pl.BlockSpec((1, D), lambda i: (0, 0)),
        pl.BlockSpec((RB, 1), lambda i: (i, 0)),
        pl.BlockSpec((D, 1), lambda i: (0, 0)),
        pl.BlockSpec((1, 1), lambda i: (0, 0)),
    ],
    out_specs=pl.BlockSpec((G, 1), lambda i: (0, 0)),
    out_shape=jax.ShapeDtypeStruct((G, 1), jnp.float32),
    scratch_shapes=[
        pltpu.VMEM((G, D), jnp.float32),
        pltpu.VMEM((G, 1), jnp.float32),
    ],
)


def kernel(x, edge_index, batch_index, W0, b0, W1, b1, W2, b2, W3, b3,
           Wfc, bfc):
    src = edge_index[0].astype(jnp.int32)
    dst = edge_index[1].astype(jnp.int32)
    pad_e = E_PAD - E
    src_p = jnp.concatenate(
        [src, (jnp.arange(pad_e, dtype=jnp.int32) % 64)])
    dst_p = jnp.concatenate(
        [dst, jnp.full((pad_e,), 1 << 29, jnp.int32)])
    xp = jnp.zeros((NP, DIN), jnp.float32).at[:N].set(x)
    bi = jnp.concatenate(
        [batch_index.astype(jnp.int32),
         jnp.full((NP - N,), G + 7, jnp.int32)]).reshape(NP, 1)
    zeros2 = jnp.zeros((128, D), jnp.float32)

    deg, plist, cnts = _part(src_p, dst_p)
    deg2 = deg.reshape(NP, 1)
    g = _g0(xp, W0, deg2)
    acc = _agg(plist, cnts, g, zeros2)
    for bprev, wnext in ((b0, W1), (b1, W2), (b2, W3)):
        g = _mid(acc, g, deg2, bprev.reshape(1, D), wnext)
        acc = _agg(plist, cnts, g, zeros2)
    return _final(acc, g, deg2, b3.reshape(1, D), bi,
                  Wfc, bfc.reshape(1, 1))


# speedup vs baseline: 36.7142x; 1.3425x over previous
import functools

import jax
import jax.numpy as jnp
from jax import lax
from jax.experimental import pallas as pl
from jax.experimental.pallas import tpu as pltpu
from jax.experimental.pallas import tpu_sc as plsc

N = 50000
NP = 50176
DIN = 12
D = 64
G = 256
HALF = 25088
ACC_ROWS = 26624
TRASH = 25088
ZROWS = ACC_ROWS // 16
VROWS = HALF // 16
E = 800000
EPT = 50176
E_PAD = EPT * 16
GE = 512
GROUPS = EPT // GE
AGE = 512
CPG = AGE // 128
CAP = 51200
NP2 = NP // 2
RB = 512
GRID = NP2 // RB

_mesh = plsc.VectorSubcoreMesh(core_axis_name="c", subcore_axis_name="s")
_sc_params = pltpu.CompilerParams(use_tc_tiling_on_sc=False,
                                  needs_layout_passes=False)


@functools.partial(
    pl.kernel,
    mesh=_mesh,
    out_type=[
        jax.ShapeDtypeStruct((NP,), jnp.float32),
        jax.ShapeDtypeStruct((32, CAP), jnp.int32),
        jax.ShapeDtypeStruct((32, 16), jnp.int32),
    ],
    scratch_types=[
        pltpu.VMEM((2, GE), jnp.int32),
        pltpu.VMEM((2, GE), jnp.int32),
        pltpu.VMEM((2, CPG, 128), jnp.int32),
        pltpu.VMEM((CAP,), jnp.int32),
        pltpu.VMEM((128,), jnp.float32),
        pltpu.VMEM((16,), jnp.int32),
        pltpu.VMEM((VROWS,), jnp.float32),
        pltpu.VMEM_SHARED((ACC_ROWS,), jnp.float32),
        pltpu.SemaphoreType.DMA((2, 2)),
        pltpu.SemaphoreType.DMA((2, CPG)),
    ],
    compiler_params=_sc_params,
)
def _part(src_hbm, dst_hbm, deg_hbm, pl_hbm, cnt_hbm,
          sstage, dstage, lstage, pbuf, ones_v, cbuf, obuf, dacc,
          sem_st, sem_d):
    c = lax.axis_index("c")
    s = lax.axis_index("s")
    wid = c * 16 + s
    lo = c * HALF
    base = s * EPT
    for kk in range(8):
        ones_v[pl.ds(kk * 16, 16)] = jnp.zeros((16,), jnp.float32)
    for kb in range(ZROWS // 128):
        pltpu.sync_copy(ones_v, dacc.at[pl.ds(s * ZROWS + kb * 128, 128)])
    for kk in range(8):
        ones_v[pl.ds(kk * 16, 16)] = jnp.full((16,), 1.0, jnp.float32)
    plsc.subcore_barrier()

    def stage(gidx, p):
        off = base + gidx * GE
        pltpu.async_copy(src_hbm.at[pl.ds(off, GE)], sstage.at[p],
                         sem_st.at[p, 0])
        pltpu.async_copy(dst_hbm.at[pl.ds(off, GE)], dstage.at[p],
                         sem_st.at[p, 1])

    def wait_stage(p):
        pltpu.make_async_copy(src_hbm.at[pl.ds(0, GE)], sstage.at[p],
                              sem_st.at[p, 0]).wait()
        pltpu.make_async_copy(dst_hbm.at[pl.ds(0, GE)], dstage.at[p],
                              sem_st.at[p, 1]).wait()

    stage(0, 0)
    stage(1, 1)

    def pair(k, cnt):
        for p in range(2):
            gidx = 2 * k + p
            wait_stage(p)
            for kb in range(CPG):
                @pl.when(k > 0)
                def _():
                    pltpu.make_async_copy(ones_v, dacc.at[lstage.at[p, kb]],
                                          sem_d.at[p, kb]).wait()
                for kk in range(8):
                    i0 = kb * 128 + kk * 16
                    d16 = dstage[p, pl.ds(i0, 16)]
                    s16 = sstage[p, pl.ds(i0, 16)]
                    inr = (d16 >= lo) & (d16 < lo + HALF)
                    loc = jnp.where(inr, d16 - lo, TRASH + (d16 & 1023))
                    lstage[p, kb, pl.ds(kk * 16, 16)] = loc
                    key = jnp.where(inr, 0, 1).astype(jnp.int32)
                    pk = s16 | (loc << 16)
                    _, vs = plsc.sort_key_val(key, pk)
                    pbuf[pl.ds(cnt, 16)] = vs
                    cnt = cnt + jnp.sum(inr.astype(jnp.int32))
                pltpu.async_copy(ones_v, dacc.at[lstage.at[p, kb]],
                                 sem_d.at[p, kb], add=True)

            @pl.when(gidx + 2 < GROUPS)
            def _():
                stage(gidx + 2, p)
        return cnt

    cnt = lax.fori_loop(0, GROUPS // 2, pair, jnp.int32(0))
    for p in range(2):
        for kb in range(CPG):
            pltpu.make_async_copy(ones_v, dacc.at[lstage.at[p, kb]],
                                  sem_d.at[p, kb]).wait()
    lane = lax.iota(jnp.int32, 16)
    for kk in range(64):
        idx = lane + kk * 16
        pbuf[pl.ds(cnt + kk * 16, 16)] = (idx % 64) | ((TRASH + (idx & 1023)) << 16)
    cnt_pad = ((cnt + 1023) // 1024) * 1024
    cbuf[pl.ds(0, 16)] = jnp.zeros((16,), jnp.int32) + cnt_pad
    pltpu.sync_copy(cbuf, cnt_hbm.at[wid])

    def flush(j, carry):
        pltpu.sync_copy(pbuf.at[pl.ds(j * 2048, 2048)],
                        pl_hbm.at[wid, pl.ds(j * 2048, 2048)])
        return carry

    lax.fori_loop(0, (cnt_pad + 2047) // 2048, flush, 0)
    plsc.subcore_barrier()
    pltpu.sync_copy(dacc.at[pl.ds(s * VROWS, VROWS)], obuf)
    pltpu.sync_copy(obuf, deg_hbm.at[pl.ds(c * HALF + s * VROWS, VROWS)])


@functools.partial(
    pl.kernel,
    mesh=_mesh,
    out_type=jax.ShapeDtypeStruct((NP, D), jnp.float32),
    scratch_types=[
        pltpu.VMEM((2, AGE), jnp.int32),
        pltpu.VMEM((2, AGE), jnp.int32),
        pltpu.VMEM((2, CPG, 128), jnp.int32),
        pltpu.VMEM((2, 128, D), jnp.float32),
        pltpu.VMEM((16,), jnp.int32),
        pltpu.VMEM_SHARED((ACC_ROWS, D), jnp.float32),
        pltpu.SemaphoreType.DMA((2, 2)),
        pltpu.SemaphoreType.DMA((2,)),
        pltpu.SemaphoreType.DMA((2,)),
    ],
    compiler_params=_sc_params,
)
def _agg(pl_hbm, cnt_hbm, g_hbm, z2_hbm, out_hbm,
         pstage, sstage, lstage, rows, cbuf, acc, sem_st, sem_g, sem_s):
    c = lax.axis_index("c")
    s = lax.axis_index("s")
    wid = c * 16 + s
    pltpu.sync_copy(cnt_hbm.at[wid], cbuf)
    npairs = jnp.max(cbuf[...]) // 1024
    pltpu.sync_copy(z2_hbm, rows.at[0])
    for kb in range(ZROWS // 128):
        pltpu.sync_copy(rows.at[0], acc.at[pl.ds(s * ZROWS + kb * 128, 128)])
    plsc.subcore_barrier()

    def stage(gidx, p):
        pltpu.async_copy(pl_hbm.at[wid, pl.ds(gidx * AGE, AGE)],
                         pstage.at[p], sem_st.at[p, 0])

    def wait_stage(p):
        pltpu.make_async_copy(pl_hbm.at[wid, pl.ds(0, AGE)], pstage.at[p],
                              sem_st.at[p, 0]).wait()

    def unpack(p):
        for kb in range(CPG):
            for kk in range(8):
                v16 = pstage[p, pl.ds(kb * 128 + kk * 16, 16)]
                sstage[p, pl.ds(kb * 128 + kk * 16, 16)] = v16 & 0xFFFF
                lstage[p, kb, pl.ds(kk * 16, 16)] = (
                    lax.shift_right_logical(v16, 16))

    def gath(p, b, slot):
        pltpu.async_copy(g_hbm.at[sstage.at[p, pl.ds(b * 128, 128)]],
                         rows.at[slot], sem_g.at[slot])

    def scat(p, b, slot):
        pltpu.async_copy(rows.at[slot], acc.at[lstage.at[p, b]],
                         sem_s.at[slot], add=True)

    def wait_scat(p, b, slot):
        pltpu.make_async_copy(rows.at[slot], acc.at[lstage.at[p, b]],
                              sem_s.at[slot]).wait()

    def wait_gath(p, b, slot):
        pltpu.make_async_copy(g_hbm.at[sstage.at[p, pl.ds(b * 128, 128)]],
                              rows.at[slot], sem_g.at[slot]).wait()

    @pl.when(npairs > 0)
    def _():
        stage(0, 0)

    def pair(k, carry):
        for idx in range(8):
            p, b, slot = idx // 4, idx % 4, idx % 2
            if idx == 0:
                wait_stage(0)
                unpack(0)
            if idx == 4:
                wait_stage(1)
                unpack(1)
            if idx < 2:
                @pl.when(k > 0)
                def _():
                    wait_scat(1, 2 + idx, idx)
            else:
                wait_scat((idx - 2) // 4, (idx - 2) % 4, slot)
            gath(p, b, slot)
            if idx == 0:
                @pl.when(k > 0)
                def _():
                    wait_gath(1, 3, 1)
                    scat(1, 3, 1)
                stage(2 * k + 1, 1)
            else:
                pi, bi = (idx - 1) // 4, (idx - 1) % 4
                wait_gath(pi, bi, 1 - slot)
                scat(pi, bi, 1 - slot)
                if idx == 4:
                    @pl.when(k + 1 < npairs)
                    def _():
                        stage(2 * k + 2, 0)
        return carry

    lax.fori_loop(0, npairs, pair, 0)

    @pl.when(npairs > 0)
    def _():
        wait_gath(1, 3, 1)
        scat(1, 3, 1)
        wait_scat(1, 2, 0)
        wait_scat(1, 3, 1)

    plsc.subcore_barrier()
    for kb in range(12):
        pltpu.sync_copy(acc.at[pl.ds(s * VROWS + kb * 128, 128)],
                        rows.at[0])
        pltpu.sync_copy(
            rows.at[0],
            out_hbm.at[pl.ds(c * HALF + s * VROWS + kb * 128, 128)])
    pltpu.sync_copy(acc.at[pl.ds(s * VROWS + 1536, 32)],
                    rows.at[0, pl.ds(0, 32)])
    pltpu.sync_copy(rows.at[0, pl.ds(0, 32)],
                    out_hbm.at[pl.ds(c * HALF + s * VROWS + 1536, 32)])


def _g0_body(x2_ref, w_ref, deg_ref, g_ref, dinv_ref):
    d = deg_ref[...] + 1.0
    d0 = jnp.broadcast_to(lax.rsqrt(d[:, 0:1]), (RB, D))
    d1 = jnp.broadcast_to(lax.rsqrt(d[:, 1:2]), (RB, D))
    dinv2 = jnp.concatenate([d0, d1], axis=1)
    w = w_ref[...]
    z = jnp.zeros((DIN, D), jnp.float32)
    wd = jnp.concatenate([jnp.concatenate([w, z], 1),
                          jnp.concatenate([z, w], 1)], 0)
    g_ref[...] = jnp.dot(x2_ref[...], wd,
                         preferred_element_type=jnp.float32) * dinv2
    dinv_ref[...] = dinv2


_g0 = pl.pallas_call(
    _g0_body,
    grid=(GRID,),
    in_specs=[
        pl.BlockSpec((RB, 2 * DIN), lambda i: (i, 0)),
        pl.BlockSpec((DIN, D), lambda i: (0, 0)),
        pl.BlockSpec((RB, 2), lambda i: (i, 0)),
    ],
    out_specs=[
        pl.BlockSpec((RB, 2 * D), lambda i: (i, 0)),
        pl.BlockSpec((RB, 2 * D), lambda i: (i, 0)),
    ],
    out_shape=[
        jax.ShapeDtypeStruct((NP2, 2 * D), jnp.float32),
        jax.ShapeDtypeStruct((NP2, 2 * D), jnp.float32),
    ],
)


def _mid_body(acc_ref, g_ref, dinv_ref, b_ref, w_ref, out_ref):
    dinv2 = dinv_ref[...]
    pre = (acc_ref[...] + g_ref[...]) * dinv2 + b_ref[...]
    xl = jnp.maximum(pre, 0.01 * pre)
    w = w_ref[...]
    z = jnp.zeros((D, D), jnp.float32)
    wd = jnp.concatenate([jnp.concatenate([w, z], 1),
                          jnp.concatenate([z, w], 1)], 0)
    out_ref[...] = jnp.dot(xl, wd,
                           preferred_element_type=jnp.float32) * dinv2


_mid = pl.pallas_call(
    _mid_body,
    grid=(GRID,),
    in_specs=[
        pl.BlockSpec((RB, 2 * D), lambda i: (i, 0)),
        pl.BlockSpec((RB, 2 * D), lambda i: (i, 0)),
        pl.BlockSpec((RB, 2 * D), lambda i: (i, 0)),
        pl.BlockSpec((1, 2 * D), lambda i: (0, 0)),
        pl.BlockSpec((D, D), lambda i: (0, 0)),
    ],
    out_specs=pl.BlockSpec((RB, 2 * D), lambda i: (i, 0)),
    out_shape=jax.ShapeDtypeStruct((NP2, 2 * D), jnp.float32),
)


def _final_body(acc_ref, g_ref, dinv_ref, b_ref, bi_ref, wfc_ref, bfc_ref,
                out_ref, sums, counts):
    i = pl.program_id(0)

    @pl.when(i == 0)
    def _():
        sums[...] = jnp.zeros_like(sums)
        counts[...] = jnp.zeros_like(counts)

    h2 = jnp.maximum((acc_ref[...] + g_ref[...]) * dinv_ref[...]
                     + b_ref[...], 0.0)
    iot = lax.broadcasted_iota(jnp.int32, (RB, G), 1)
    oh_e = (bi_ref[...][:, 0:1] == iot).astype(jnp.float32)
    oh_o = (bi_ref[...][:, 1:2] == iot).astype(jnp.float32)
    dn = (((0,), (0,)), ((), ()))
    ones = jnp.ones((RB, 1), jnp.float32)
    sums[...] += (
        lax.dot_general(oh_e, h2[:, :D], dn,
                        preferred_element_type=jnp.float32)
        + lax.dot_general(oh_o, h2[:, D:], dn,
                          preferred_element_type=jnp.float32))
    counts[...] += (
        lax.dot_general(oh_e, ones, dn, preferred_element_type=jnp.float32)
        + lax.dot_general(oh_o, ones, dn,
                          preferred_element_type=jnp.float32))

    @pl.when(i == GRID - 1)
    def _():
        mean = sums[...] / jnp.maximum(counts[...], 1.0)
        z = jnp.dot(mean, wfc_ref[...],
                    preferred_element_type=jnp.float32) + bfc_ref[...]
        out_ref[...] = jax.nn.sigmoid(z)


_final = pl.pallas_call(
    _final_body,
    grid=(GRID,),
    in_specs=[
        pl.BlockSpec((RB, 2 * D), lambda i: (i, 0)),
        pl.BlockSpec((RB, 2 * D), lambda i: (i, 0)),
        pl.BlockSpec((RB, 2 * D), lambda i: (i, 0)),
        pl.BlockSpec((1, 2 * D), lambda i: (0, 0)),
        pl.BlockSpec((RB, 2), lambda i: (i, 0)),
        pl.BlockSpec((D, 1), lambda i: (0, 0)),
        pl.BlockSpec((1, 1), lambda i: (0, 0)),
    ],
    out_specs=pl.BlockSpec((G, 1), lambda i: (0, 0)),
    out_shape=jax.ShapeDtypeStruct((G, 1), jnp.float32),
    scratch_shapes=[
        pltpu.VMEM((G, D), jnp.float32),
        pltpu.VMEM((G, 1), jnp.float32),
    ],
)


def kernel(x, edge_index, batch_index, W0, b0, W1, b1, W2, b2, W3, b3,
           Wfc, bfc):
    src = edge_index[0].astype(jnp.int32)
    dst = edge_index[1].astype(jnp.int32)
    pad_e = E_PAD - E
    src_p = jnp.concatenate(
        [src, (jnp.arange(pad_e, dtype=jnp.int32) % 64)])
    dst_p = jnp.concatenate(
        [dst, jnp.full((pad_e,), 1 << 29, jnp.int32)])
    xp = jnp.zeros((NP, DIN), jnp.float32).at[:N].set(x)
    bi = jnp.concatenate(
        [batch_index.astype(jnp.int32),
         jnp.full((NP - N,), G + 7, jnp.int32)]).reshape(NP, 1)
    zeros2 = jnp.zeros((128, D), jnp.float32)

    deg, plist, cnts = _part(src_p, dst_p)
    x2 = xp.reshape(NP2, 2 * DIN)
    deg2p = deg.reshape(NP2, 2)
    batch2 = bi.reshape(NP2, 2)
    g2, dinv2 = _g0(x2, W0, deg2p)
    acc = _agg(plist, cnts, g2.reshape(NP, D), zeros2)
    for bprev, wnext in ((b0, W1), (b1, W2), (b2, W3)):
        b2 = jnp.concatenate([bprev, bprev]).reshape(1, 2 * D)
        g2 = _mid(acc.reshape(NP2, 2 * D), g2, dinv2, b2, wnext)
        acc = _agg(plist, cnts, g2.reshape(NP, D), zeros2)
    b32 = jnp.concatenate([b3, b3]).reshape(1, 2 * D)
    return _final(acc.reshape(NP2, 2 * D), g2, dinv2, b32, batch2,
                  Wfc, bfc.reshape(1, 1))


# speedup vs baseline: 41.0257x; 1.1174x over previous
import functools

import jax
import jax.numpy as jnp
from jax import lax
from jax.experimental import pallas as pl
from jax.experimental.pallas import tpu as pltpu
from jax.experimental.pallas import tpu_sc as plsc

N = 50000
NP = 50176
DIN = 12
D = 64
G = 256
HALF = 25088
ACC_ROWS = 25216
TRASH = 25088
ZROWS = ACC_ROWS // 16
VROWS = HALF // 16
E = 800000
EPT = 50176
E_PAD = EPT * 16
GE = 512
GROUPS = EPT // GE
AGE = 512
CPG = AGE // 128
CAP = 52224
NP2 = NP // 2
RB = 512
GRID = NP2 // RB

_mesh = plsc.VectorSubcoreMesh(core_axis_name="c", subcore_axis_name="s")
_sc_params = pltpu.CompilerParams(use_tc_tiling_on_sc=False,
                                  needs_layout_passes=False)


@functools.partial(
    pl.kernel,
    mesh=_mesh,
    out_type=[
        jax.ShapeDtypeStruct((NP,), jnp.float32),
        jax.ShapeDtypeStruct((32, CAP), jnp.int32),
        jax.ShapeDtypeStruct((32, 16), jnp.int32),
    ],
    scratch_types=[
        pltpu.VMEM((2, GE), jnp.int32),
        pltpu.VMEM((2, GE), jnp.int32),
        pltpu.VMEM((2, CPG, 128), jnp.int32),
        pltpu.VMEM((CAP,), jnp.int32),
        pltpu.VMEM((128,), jnp.float32),
        pltpu.VMEM((16,), jnp.int32),
        pltpu.VMEM((VROWS,), jnp.float32),
        pltpu.VMEM_SHARED((ACC_ROWS,), jnp.float32),
        pltpu.SemaphoreType.DMA((2, 2)),
        pltpu.SemaphoreType.DMA((2, CPG)),
    ],
    compiler_params=_sc_params,
)
def _part(src_hbm, dst_hbm, deg_hbm, pl_hbm, cnt_hbm,
          sstage, dstage, lstage, pbuf, ones_v, cbuf, obuf, dacc,
          sem_st, sem_d):
    c = lax.axis_index("c")
    s = lax.axis_index("s")
    wid = c * 16 + s
    lo = c * HALF
    base = s * EPT
    for kk in range(8):
        ones_v[pl.ds(kk * 16, 16)] = jnp.zeros((16,), jnp.float32)
    for kb in range(12):
        pltpu.sync_copy(ones_v, dacc.at[pl.ds(s * ZROWS + kb * 128, 128)])
    pltpu.sync_copy(ones_v.at[pl.ds(0, 40)],
                    dacc.at[pl.ds(s * ZROWS + 1536, 40)])
    for kk in range(8):
        ones_v[pl.ds(kk * 16, 16)] = jnp.full((16,), 1.0, jnp.float32)
    plsc.subcore_barrier()

    def stage(gidx, p):
        off = base + gidx * GE
        pltpu.async_copy(src_hbm.at[pl.ds(off, GE)], sstage.at[p],
                         sem_st.at[p, 0])
        pltpu.async_copy(dst_hbm.at[pl.ds(off, GE)], dstage.at[p],
                         sem_st.at[p, 1])

    def wait_stage(p):
        pltpu.make_async_copy(src_hbm.at[pl.ds(0, GE)], sstage.at[p],
                              sem_st.at[p, 0]).wait()
        pltpu.make_async_copy(dst_hbm.at[pl.ds(0, GE)], dstage.at[p],
                              sem_st.at[p, 1]).wait()

    stage(0, 0)
    stage(1, 1)

    def pair(k, cnt):
        for p in range(2):
            gidx = 2 * k + p
            wait_stage(p)
            for kb in range(CPG):
                @pl.when(k > 0)
                def _():
                    pltpu.make_async_copy(ones_v, dacc.at[lstage.at[p, kb]],
                                          sem_d.at[p, kb]).wait()
                for kk in range(8):
                    i0 = kb * 128 + kk * 16
                    d16 = dstage[p, pl.ds(i0, 16)]
                    s16 = sstage[p, pl.ds(i0, 16)]
                    inr = (d16 >= lo) & (d16 < lo + HALF)
                    loc = jnp.where(inr, d16 - lo, TRASH + (d16 & 127))
                    lstage[p, kb, pl.ds(kk * 16, 16)] = loc
                    key = jnp.where(inr, 0, 1).astype(jnp.int32)
                    pk = s16 | (loc << 16)
                    _, vs = plsc.sort_key_val(key, pk)
                    pbuf[pl.ds(cnt, 16)] = vs
                    cnt = cnt + jnp.sum(inr.astype(jnp.int32))
                pltpu.async_copy(ones_v, dacc.at[lstage.at[p, kb]],
                                 sem_d.at[p, kb], add=True)

            @pl.when(gidx + 2 < GROUPS)
            def _():
                stage(gidx + 2, p)
        return cnt

    cnt = lax.fori_loop(0, GROUPS // 2, pair, jnp.int32(0))
    for p in range(2):
        for kb in range(CPG):
            pltpu.make_async_copy(ones_v, dacc.at[lstage.at[p, kb]],
                                  sem_d.at[p, kb]).wait()
    lane = lax.iota(jnp.int32, 16)
    for kk in range(96):
        idx = lane + kk * 16
        pbuf[pl.ds(cnt + kk * 16, 16)] = (idx % 64) | ((TRASH + (idx & 127)) << 16)
    cnt_pad = ((cnt + 1535) // 1536) * 1536
    cbuf[pl.ds(0, 16)] = jnp.zeros((16,), jnp.int32) + cnt_pad
    pltpu.sync_copy(cbuf, cnt_hbm.at[wid])

    def flush(j, carry):
        pltpu.sync_copy(pbuf.at[pl.ds(j * 2048, 2048)],
                        pl_hbm.at[wid, pl.ds(j * 2048, 2048)])
        return carry

    lax.fori_loop(0, (cnt_pad + 2047) // 2048, flush, 0)
    plsc.subcore_barrier()
    pltpu.sync_copy(dacc.at[pl.ds(s * VROWS, VROWS)], obuf)
    pltpu.sync_copy(obuf, deg_hbm.at[pl.ds(c * HALF + s * VROWS, VROWS)])


@functools.partial(
    pl.kernel,
    mesh=_mesh,
    out_type=jax.ShapeDtypeStruct((NP, D), jnp.float32),
    scratch_types=[
        pltpu.VMEM((3, AGE), jnp.int32),
        pltpu.VMEM((3, AGE), jnp.int32),
        pltpu.VMEM((3, CPG, 128), jnp.int32),
        pltpu.VMEM((3, 128, D), jnp.float32),
        pltpu.VMEM((16,), jnp.int32),
        pltpu.VMEM_SHARED((ACC_ROWS, D), jnp.float32),
        pltpu.SemaphoreType.DMA((3,)),
        pltpu.SemaphoreType.DMA((3,)),
        pltpu.SemaphoreType.DMA((3,)),
    ],
    compiler_params=_sc_params,
)
def _agg(pl_hbm, cnt_hbm, g_hbm, z2_hbm, out_hbm,
         pstage, sstage, lstage, rows, cbuf, acc, sem_st, sem_g, sem_s):
    c = lax.axis_index("c")
    s = lax.axis_index("s")
    wid = c * 16 + s
    pltpu.sync_copy(cnt_hbm.at[wid], cbuf)
    niter = jnp.max(cbuf[...]) // 1536
    pltpu.sync_copy(z2_hbm, rows.at[0])
    for kb in range(12):
        pltpu.sync_copy(rows.at[0], acc.at[pl.ds(s * ZROWS + kb * 128, 128)])
    pltpu.sync_copy(rows.at[0, pl.ds(0, 40)],
                    acc.at[pl.ds(s * ZROWS + 1536, 40)])
    plsc.subcore_barrier()

    def stage(gidx, p):
        pltpu.async_copy(pl_hbm.at[wid, pl.ds(gidx * AGE, AGE)],
                         pstage.at[p], sem_st.at[p])

    def wait_stage(p):
        pltpu.make_async_copy(pl_hbm.at[wid, pl.ds(0, AGE)], pstage.at[p],
                              sem_st.at[p]).wait()

    def gath(p, b, slot):
        pltpu.async_copy(g_hbm.at[sstage.at[p, pl.ds(b * 128, 128)]],
                         rows.at[slot], sem_g.at[slot])

    def scat(p, b, slot):
        pltpu.async_copy(rows.at[slot], acc.at[lstage.at[p, b]],
                         sem_s.at[slot], add=True)

    def wait_scat(p, b, slot):
        pltpu.make_async_copy(rows.at[slot], acc.at[lstage.at[p, b]],
                              sem_s.at[slot]).wait()

    def wait_gath(p, b, slot):
        pltpu.make_async_copy(g_hbm.at[sstage.at[p, pl.ds(b * 128, 128)]],
                              rows.at[slot], sem_g.at[slot]).wait()

    def unpack(p):
        for kb in range(CPG):
            for kk in range(8):
                v16 = pstage[p, pl.ds(kb * 128 + kk * 16, 16)]
                sstage[p, pl.ds(kb * 128 + kk * 16, 16)] = v16 & 0xFFFF
                lstage[p, kb, pl.ds(kk * 16, 16)] = (
                    lax.shift_right_logical(v16, 16))

    @pl.when(niter > 0)
    def _():
        stage(0, 0)
        stage(1, 1)

    def titer(k, carry):
        for j in range(12):
            p, b, slot = j // 4, j % 4, j % 3
            if j == 0:
                wait_stage(0)
                unpack(0)
            if j == 4:
                wait_stage(1)
                unpack(1)
            if j == 8:
                wait_stage(2)
                unpack(2)
            if j < 3:
                @pl.when(k > 0)
                def _():
                    wait_scat(2, (9 + j) % 4, slot)
            else:
                wait_scat((j - 3) // 4, (j - 3) % 4, slot)
            gath(p, b, slot)
            if j < 2:
                @pl.when(k > 0)
                def _():
                    wait_gath(2, 2 + j, (10 + j) % 3)
                    scat(2, 2 + j, (10 + j) % 3)
            else:
                jj = j - 2
                wait_gath(jj // 4, jj % 4, jj % 3)
                scat(jj // 4, jj % 4, jj % 3)
            if j == 1:
                stage(3 * k + 2, 2)
            if j == 5:
                @pl.when(k + 1 < niter)
                def _():
                    stage(3 * k + 3, 0)
            if j == 9:
                @pl.when(k + 1 < niter)
                def _():
                    stage(3 * k + 4, 1)
        return carry

    lax.fori_loop(0, niter, titer, 0)

    @pl.when(niter > 0)
    def _():
        wait_gath(2, 2, 1)
        scat(2, 2, 1)
        wait_gath(2, 3, 2)
        scat(2, 3, 2)
        wait_scat(2, 1, 0)
        wait_scat(2, 2, 1)
        wait_scat(2, 3, 2)

    plsc.subcore_barrier()
    for kb in range(12):
        pltpu.sync_copy(acc.at[pl.ds(s * VROWS + kb * 128, 128)],
                        rows.at[0])
        pltpu.sync_copy(
            rows.at[0],
            out_hbm.at[pl.ds(c * HALF + s * VROWS + kb * 128, 128)])
    pltpu.sync_copy(acc.at[pl.ds(s * VROWS + 1536, 32)],
                    rows.at[0, pl.ds(0, 32)])
    pltpu.sync_copy(rows.at[0, pl.ds(0, 32)],
                    out_hbm.at[pl.ds(c * HALF + s * VROWS + 1536, 32)])


def _g0_body(x2_ref, w_ref, deg_ref, g_ref, dinv_ref):
    d = deg_ref[...] + 1.0
    d0 = jnp.broadcast_to(lax.rsqrt(d[:, 0:1]), (RB, D))
    d1 = jnp.broadcast_to(lax.rsqrt(d[:, 1:2]), (RB, D))
    dinv2 = jnp.concatenate([d0, d1], axis=1)
    w = w_ref[...]
    z = jnp.zeros((DIN, D), jnp.float32)
    wd = jnp.concatenate([jnp.concatenate([w, z], 1),
                          jnp.concatenate([z, w], 1)], 0)
    g_ref[...] = jnp.dot(x2_ref[...], wd,
                         preferred_element_type=jnp.float32) * dinv2
    dinv_ref[...] = dinv2


_g0 = pl.pallas_call(
    _g0_body,
    grid=(GRID,),
    in_specs=[
        pl.BlockSpec((RB, 2 * DIN), lambda i: (i, 0)),
        pl.BlockSpec((DIN, D), lambda i: (0, 0)),
        pl.BlockSpec((RB, 2), lambda i: (i, 0)),
    ],
    out_specs=[
        pl.BlockSpec((RB, 2 * D), lambda i: (i, 0)),
        pl.BlockSpec((RB, 2 * D), lambda i: (i, 0)),
    ],
    out_shape=[
        jax.ShapeDtypeStruct((NP2, 2 * D), jnp.float32),
        jax.ShapeDtypeStruct((NP2, 2 * D), jnp.float32),
    ],
)


def _mid_body(acc_ref, g_ref, dinv_ref, b_ref, w_ref, out_ref):
    dinv2 = dinv_ref[...]
    pre = (acc_ref[...] + g_ref[...]) * dinv2 + b_ref[...]
    xl = jnp.maximum(pre, 0.01 * pre)
    w = w_ref[...]
    z = jnp.zeros((D, D), jnp.float32)
    wd = jnp.concatenate([jnp.concatenate([w, z], 1),
                          jnp.concatenate([z, w], 1)], 0)
    out_ref[...] = jnp.dot(xl, wd,
                           preferred_element_type=jnp.float32) * dinv2


_mid = pl.pallas_call(
    _mid_body,
    grid=(GRID,),
    in_specs=[
        pl.BlockSpec((RB, 2 * D), lambda i: (i, 0)),
        pl.BlockSpec((RB, 2 * D), lambda i: (i, 0)),
        pl.BlockSpec((RB, 2 * D), lambda i: (i, 0)),
        pl.BlockSpec((1, 2 * D), lambda i: (0, 0)),
        pl.BlockSpec((D, D), lambda i: (0, 0)),
    ],
    out_specs=pl.BlockSpec((RB, 2 * D), lambda i: (i, 0)),
    out_shape=jax.ShapeDtypeStruct((NP2, 2 * D), jnp.float32),
)


def _final_body(acc_ref, g_ref, dinv_ref, b_ref, bi_ref, wfc_ref, bfc_ref,
                out_ref, sums, counts):
    i = pl.program_id(0)

    @pl.when(i == 0)
    def _():
        sums[...] = jnp.zeros_like(sums)
        counts[...] = jnp.zeros_like(counts)

    h2 = jnp.maximum((acc_ref[...] + g_ref[...]) * dinv_ref[...]
                     + b_ref[...], 0.0)
    iot = lax.broadcasted_iota(jnp.int32, (RB, G), 1)
    oh_e = (bi_ref[...][:, 0:1] == iot).astype(jnp.float32)
    oh_o = (bi_ref[...][:, 1:2] == iot).astype(jnp.float32)
    dn = (((0,), (0,)), ((), ()))
    ones = jnp.ones((RB, 1), jnp.float32)
    sums[...] += (
        lax.dot_general(oh_e, h2[:, :D], dn,
                        preferred_element_type=jnp.float32)
        + lax.dot_general(oh_o, h2[:, D:], dn,
                          preferred_element_type=jnp.float32))
    counts[...] += (
        lax.dot_general(oh_e, ones, dn, preferred_element_type=jnp.float32)
        + lax.dot_general(oh_o, ones, dn,
                          preferred_element_type=jnp.float32))

    @pl.when(i == GRID - 1)
    def _():
        mean = sums[...] / jnp.maximum(counts[...], 1.0)
        z = jnp.dot(mean, wfc_ref[...],
                    preferred_element_type=jnp.float32) + bfc_ref[...]
        out_ref[...] = jax.nn.sigmoid(z)


_final = pl.pallas_call(
    _final_body,
    grid=(GRID,),
    in_specs=[
        pl.BlockSpec((RB, 2 * D), lambda i: (i, 0)),
        pl.BlockSpec((RB, 2 * D), lambda i: (i, 0)),
        pl.BlockSpec((RB, 2 * D), lambda i: (i, 0)),
        pl.BlockSpec((1, 2 * D), lambda i: (0, 0)),
        pl.BlockSpec((RB, 2), lambda i: (i, 0)),
        pl.BlockSpec((D, 1), lambda i: (0, 0)),
        pl.BlockSpec((1, 1), lambda i: (0, 0)),
    ],
    out_specs=pl.BlockSpec((G, 1), lambda i: (0, 0)),
    out_shape=jax.ShapeDtypeStruct((G, 1), jnp.float32),
    scratch_shapes=[
        pltpu.VMEM((G, D), jnp.float32),
        pltpu.VMEM((G, 1), jnp.float32),
    ],
)


def kernel(x, edge_index, batch_index, W0, b0, W1, b1, W2, b2, W3, b3,
           Wfc, bfc):
    src = edge_index[0].astype(jnp.int32)
    dst = edge_index[1].astype(jnp.int32)
    pad_e = E_PAD - E
    src_p = jnp.concatenate(
        [src, (jnp.arange(pad_e, dtype=jnp.int32) % 64)])
    dst_p = jnp.concatenate(
        [dst, jnp.full((pad_e,), 1 << 29, jnp.int32)])
    xp = jnp.zeros((NP, DIN), jnp.float32).at[:N].set(x)
    bi = jnp.concatenate(
        [batch_index.astype(jnp.int32),
         jnp.full((NP - N,), G + 7, jnp.int32)]).reshape(NP, 1)
    zeros2 = jnp.zeros((128, D), jnp.float32)

    deg, plist, cnts = _part(src_p, dst_p)
    x2 = xp.reshape(NP2, 2 * DIN)
    deg2p = deg.reshape(NP2, 2)
    batch2 = bi.reshape(NP2, 2)
    g2, dinv2 = _g0(x2, W0, deg2p)
    acc = _agg(plist, cnts, g2.reshape(NP, D), zeros2)
    for bprev, wnext in ((b0, W1), (b1, W2), (b2, W3)):
        b2 = jnp.concatenate([bprev, bprev]).reshape(1, 2 * D)
        g2 = _mid(acc.reshape(NP2, 2 * D), g2, dinv2, b2, wnext)
        acc = _agg(plist, cnts, g2.reshape(NP, D), zeros2)
    b32 = jnp.concatenate([b3, b3]).reshape(1, 2 * D)
    return _final(acc.reshape(NP2, 2 * D), g2, dinv2, b32, batch2,
                  Wfc, bfc.reshape(1, 1))


# speedup vs baseline: 42.1553x; 1.0275x over previous
import functools

import jax
import jax.numpy as jnp
from jax import lax
from jax.experimental import pallas as pl
from jax.experimental.pallas import tpu as pltpu
from jax.experimental.pallas import tpu_sc as plsc

N = 50000
NP = 50176
DIN = 12
D = 64
G = 256
HALF = 25088
ACC_ROWS = 25216
TRASH = 25088
ZROWS = ACC_ROWS // 16
VROWS = HALF // 16
E = 800000
EPT = 50176
E_PAD = EPT * 16
GE = 512
GROUPS = EPT // GE
AGE = 512
CPG = AGE // 128
CAP = 52224
NP2 = NP // 2
RB = 512
GRID = NP2 // RB

_mesh = plsc.VectorSubcoreMesh(core_axis_name="c", subcore_axis_name="s")
_sc_params = pltpu.CompilerParams(use_tc_tiling_on_sc=False,
                                  needs_layout_passes=False)


@functools.partial(
    pl.kernel,
    mesh=_mesh,
    out_type=[
        jax.ShapeDtypeStruct((NP,), jnp.float32),
        jax.ShapeDtypeStruct((32, CAP), jnp.int32),
        jax.ShapeDtypeStruct((32, 16), jnp.int32),
    ],
    scratch_types=[
        pltpu.VMEM((2, GE), jnp.int32),
        pltpu.VMEM((2, GE), jnp.int32),
        pltpu.VMEM((2, CPG, 128), jnp.int32),
        pltpu.VMEM((CAP,), jnp.int32),
        pltpu.VMEM((128,), jnp.float32),
        pltpu.VMEM((16,), jnp.int32),
        pltpu.VMEM((VROWS,), jnp.float32),
        pltpu.VMEM_SHARED((ACC_ROWS,), jnp.float32),
        pltpu.SemaphoreType.DMA((2, 2)),
        pltpu.SemaphoreType.DMA((2, CPG)),
    ],
    compiler_params=_sc_params,
)
def _part(src_hbm, dst_hbm, deg_hbm, pl_hbm, cnt_hbm,
          sstage, dstage, lstage, pbuf, ones_v, cbuf, obuf, dacc,
          sem_st, sem_d):
    c = lax.axis_index("c")
    s = lax.axis_index("s")
    wid = c * 16 + s
    lo = c * HALF
    base = s * EPT
    for kk in range(8):
        ones_v[pl.ds(kk * 16, 16)] = jnp.zeros((16,), jnp.float32)
    for kb in range(12):
        pltpu.sync_copy(ones_v, dacc.at[pl.ds(s * ZROWS + kb * 128, 128)])
    pltpu.sync_copy(ones_v.at[pl.ds(0, 40)],
                    dacc.at[pl.ds(s * ZROWS + 1536, 40)])
    for kk in range(8):
        ones_v[pl.ds(kk * 16, 16)] = jnp.full((16,), 1.0, jnp.float32)
    plsc.subcore_barrier()

    def stage(gidx, p):
        off = base + gidx * GE
        pltpu.async_copy(src_hbm.at[pl.ds(off, GE)], sstage.at[p],
                         sem_st.at[p, 0])
        pltpu.async_copy(dst_hbm.at[pl.ds(off, GE)], dstage.at[p],
                         sem_st.at[p, 1])

    def wait_stage(p):
        pltpu.make_async_copy(src_hbm.at[pl.ds(0, GE)], sstage.at[p],
                              sem_st.at[p, 0]).wait()
        pltpu.make_async_copy(dst_hbm.at[pl.ds(0, GE)], dstage.at[p],
                              sem_st.at[p, 1]).wait()

    stage(0, 0)
    stage(1, 1)

    def pair(k, cnt):
        for p in range(2):
            gidx = 2 * k + p
            wait_stage(p)
            for kb in range(CPG):
                @pl.when(k > 0)
                def _():
                    pltpu.make_async_copy(ones_v, dacc.at[lstage.at[p, kb]],
                                          sem_d.at[p, kb]).wait()
                for kk in range(8):
                    i0 = kb * 128 + kk * 16
                    d16 = dstage[p, pl.ds(i0, 16)]
                    s16 = sstage[p, pl.ds(i0, 16)]
                    inr = (d16 >= lo) & (d16 < lo + HALF)
                    loc = jnp.where(inr, d16 - lo, TRASH + (d16 & 127))
                    lstage[p, kb, pl.ds(kk * 16, 16)] = loc
                    key = jnp.where(inr, 0, 1).astype(jnp.int32)
                    pk = s16 | (loc << 16)
                    _, vs = plsc.sort_key_val(key, pk)
                    pbuf[pl.ds(cnt, 16)] = vs
                    cnt = cnt + jnp.sum(inr.astype(jnp.int32))
                pltpu.async_copy(ones_v, dacc.at[lstage.at[p, kb]],
                                 sem_d.at[p, kb], add=True)

            @pl.when(gidx + 2 < GROUPS)
            def _():
                stage(gidx + 2, p)
        return cnt

    cnt = lax.fori_loop(0, GROUPS // 2, pair, jnp.int32(0))
    for p in range(2):
        for kb in range(CPG):
            pltpu.make_async_copy(ones_v, dacc.at[lstage.at[p, kb]],
                                  sem_d.at[p, kb]).wait()
    lane = lax.iota(jnp.int32, 16)
    for kk in range(96):
        idx = lane + kk * 16
        pbuf[pl.ds(cnt + kk * 16, 16)] = (idx % 64) | ((TRASH + (idx & 127)) << 16)
    cnt_pad = ((cnt + 1535) // 1536) * 1536
    cbuf[pl.ds(0, 16)] = jnp.zeros((16,), jnp.int32) + cnt_pad
    pltpu.sync_copy(cbuf, cnt_hbm.at[wid])

    def flush(j, carry):
        pltpu.sync_copy(pbuf.at[pl.ds(j * 2048, 2048)],
                        pl_hbm.at[wid, pl.ds(j * 2048, 2048)])
        return carry

    lax.fori_loop(0, (cnt_pad + 2047) // 2048, flush, 0)
    plsc.subcore_barrier()
    pltpu.sync_copy(dacc.at[pl.ds(s * VROWS, VROWS)], obuf)
    pltpu.sync_copy(obuf, deg_hbm.at[pl.ds(c * HALF + s * VROWS, VROWS)])


def _mk_agg(d):
    @functools.partial(
        pl.kernel,
        mesh=_mesh,
        out_type=jax.ShapeDtypeStruct((NP, d), jnp.float32),
        scratch_types=[
            pltpu.VMEM((3, AGE), jnp.int32),
            pltpu.VMEM((3, AGE), jnp.int32),
            pltpu.VMEM((3, CPG, 128), jnp.int32),
            pltpu.VMEM((3, 128, d), jnp.float32),
            pltpu.VMEM((16,), jnp.int32),
            pltpu.VMEM_SHARED((ACC_ROWS, d), jnp.float32),
            pltpu.SemaphoreType.DMA((3,)),
            pltpu.SemaphoreType.DMA((3,)),
            pltpu.SemaphoreType.DMA((3,)),
        ],
        compiler_params=_sc_params,
    )
    def _aggd(pl_hbm, cnt_hbm, g_hbm, z2_hbm, out_hbm,
             pstage, sstage, lstage, rows, cbuf, acc, sem_st, sem_g, sem_s):
        c = lax.axis_index("c")
        s = lax.axis_index("s")
        wid = c * 16 + s
        pltpu.sync_copy(cnt_hbm.at[wid], cbuf)
        niter = jnp.max(cbuf[...]) // 1536
        pltpu.sync_copy(z2_hbm, rows.at[0])
        for kb in range(12):
            pltpu.sync_copy(rows.at[0], acc.at[pl.ds(s * ZROWS + kb * 128, 128)])
        pltpu.sync_copy(rows.at[0, pl.ds(0, 40)],
                        acc.at[pl.ds(s * ZROWS + 1536, 40)])
        plsc.subcore_barrier()

        def stage(gidx, p):
            pltpu.async_copy(pl_hbm.at[wid, pl.ds(gidx * AGE, AGE)],
                             pstage.at[p], sem_st.at[p])

        def wait_stage(p):
            pltpu.make_async_copy(pl_hbm.at[wid, pl.ds(0, AGE)], pstage.at[p],
                                  sem_st.at[p]).wait()

        def gath(p, b, slot):
            pltpu.async_copy(g_hbm.at[sstage.at[p, pl.ds(b * 128, 128)]],
                             rows.at[slot], sem_g.at[slot])

        def scat(p, b, slot):
            pltpu.async_copy(rows.at[slot], acc.at[lstage.at[p, b]],
                             sem_s.at[slot], add=True)

        def wait_scat(p, b, slot):
            pltpu.make_async_copy(rows.at[slot], acc.at[lstage.at[p, b]],
                                  sem_s.at[slot]).wait()

        def wait_gath(p, b, slot):
            pltpu.make_async_copy(g_hbm.at[sstage.at[p, pl.ds(b * 128, 128)]],
                                  rows.at[slot], sem_g.at[slot]).wait()

        def unpack(p):
            for kb in range(CPG):
                for kk in range(8):
                    v16 = pstage[p, pl.ds(kb * 128 + kk * 16, 16)]
                    sstage[p, pl.ds(kb * 128 + kk * 16, 16)] = v16 & 0xFFFF
                    lstage[p, kb, pl.ds(kk * 16, 16)] = (
                        lax.shift_right_logical(v16, 16))

        @pl.when(niter > 0)
        def _():
            stage(0, 0)
            stage(1, 1)

        def titer(k, carry):
            for j in range(12):
                p, b, slot = j // 4, j % 4, j % 3
                if j == 0:
                    wait_stage(0)
                    unpack(0)
                if j == 4:
                    wait_stage(1)
                    unpack(1)
                if j == 8:
                    wait_stage(2)
                    unpack(2)
                if j < 3:
                    @pl.when(k > 0)
                    def _():
                        wait_scat(2, (9 + j) % 4, slot)
                else:
                    wait_scat((j - 3) // 4, (j - 3) % 4, slot)
                gath(p, b, slot)
                if j < 2:
                    @pl.when(k > 0)
                    def _():
                        wait_gath(2, 2 + j, (10 + j) % 3)
                        scat(2, 2 + j, (10 + j) % 3)
                else:
                    jj = j - 2
                    wait_gath(jj // 4, jj % 4, jj % 3)
                    scat(jj // 4, jj % 4, jj % 3)
                if j == 1:
                    stage(3 * k + 2, 2)
                if j == 5:
                    @pl.when(k + 1 < niter)
                    def _():
                        stage(3 * k + 3, 0)
                if j == 9:
                    @pl.when(k + 1 < niter)
                    def _():
                        stage(3 * k + 4, 1)
            return carry

        lax.fori_loop(0, niter, titer, 0)

        @pl.when(niter > 0)
        def _():
            wait_gath(2, 2, 1)
            scat(2, 2, 1)
            wait_gath(2, 3, 2)
            scat(2, 3, 2)
            wait_scat(2, 1, 0)
            wait_scat(2, 2, 1)
            wait_scat(2, 3, 2)

        plsc.subcore_barrier()
        for kb in range(12):
            pltpu.sync_copy(acc.at[pl.ds(s * VROWS + kb * 128, 128)],
                            rows.at[0])
            pltpu.sync_copy(
                rows.at[0],
                out_hbm.at[pl.ds(c * HALF + s * VROWS + kb * 128, 128)])
        pltpu.sync_copy(acc.at[pl.ds(s * VROWS + 1536, 32)],
                        rows.at[0, pl.ds(0, 32)])
        pltpu.sync_copy(rows.at[0, pl.ds(0, 32)],
                        out_hbm.at[pl.ds(c * HALF + s * VROWS + 1536, 32)])


    return _aggd


_agg = _mk_agg(D)
_agg16 = _mk_agg(16)


def _u_body(x2_ref, deg_ref, u_ref, dinv_ref):
    d = deg_ref[...] + 1.0
    d0 = lax.rsqrt(d[:, 0:1])
    d1 = lax.rsqrt(d[:, 1:2])
    dinv_ref[...] = jnp.concatenate(
        [jnp.broadcast_to(d0, (RB, D)), jnp.broadcast_to(d1, (RB, D))], 1)
    x2 = x2_ref[...]
    z4 = jnp.zeros((RB, 4), jnp.float32)
    u_ref[...] = jnp.concatenate(
        [x2[:, :DIN] * jnp.broadcast_to(d0, (RB, DIN)), z4,
         x2[:, DIN:] * jnp.broadcast_to(d1, (RB, DIN)), z4], 1)


_u = pl.pallas_call(
    _u_body,
    grid=(GRID,),
    in_specs=[
        pl.BlockSpec((RB, 2 * DIN), lambda i: (i, 0)),
        pl.BlockSpec((RB, 2), lambda i: (i, 0)),
    ],
    out_specs=[
        pl.BlockSpec((RB, 32), lambda i: (i, 0)),
        pl.BlockSpec((RB, 2 * D), lambda i: (i, 0)),
    ],
    out_shape=[
        jax.ShapeDtypeStruct((NP2, 32), jnp.float32),
        jax.ShapeDtypeStruct((NP2, 2 * D), jnp.float32),
    ],
)


def _mid0_body(acc_ref, u_ref, dinv_ref, b_ref, w0_ref, w1_ref, out_ref):
    dinv2 = dinv_ref[...]
    a_e = (acc_ref[...][:, :16] + u_ref[...][:, :16]) * dinv2[:, 0:16]
    a_o = (acc_ref[...][:, 16:] + u_ref[...][:, 16:]) * dinv2[:, D:D + 16]
    a = jnp.concatenate([a_e, a_o], 1)
    wp = jnp.concatenate([w0_ref[...], jnp.zeros((4, D), jnp.float32)], 0)
    z16 = jnp.zeros((16, D), jnp.float32)
    wd0 = jnp.concatenate([jnp.concatenate([wp, z16], 1),
                           jnp.concatenate([z16, wp], 1)], 0)
    pre = jnp.dot(a, wd0, preferred_element_type=jnp.float32) + b_ref[...]
    x1 = jnp.maximum(pre, 0.01 * pre)
    w1 = w1_ref[...]
    z = jnp.zeros((D, D), jnp.float32)
    wd1 = jnp.concatenate([jnp.concatenate([w1, z], 1),
                           jnp.concatenate([z, w1], 1)], 0)
    out_ref[...] = jnp.dot(x1, wd1,
                           preferred_element_type=jnp.float32) * dinv2


_mid0 = pl.pallas_call(
    _mid0_body,
    grid=(GRID,),
    in_specs=[
        pl.BlockSpec((RB, 32), lambda i: (i, 0)),
        pl.BlockSpec((RB, 32), lambda i: (i, 0)),
        pl.BlockSpec((RB, 2 * D), lambda i: (i, 0)),
        pl.BlockSpec((1, 2 * D), lambda i: (0, 0)),
        pl.BlockSpec((DIN, D), lambda i: (0, 0)),
        pl.BlockSpec((D, D), lambda i: (0, 0)),
    ],
    out_specs=pl.BlockSpec((RB, 2 * D), lambda i: (i, 0)),
    out_shape=jax.ShapeDtypeStruct((NP2, 2 * D), jnp.float32),
)


def _mid_body(acc_ref, g_ref, dinv_ref, b_ref, w_ref, out_ref):
    dinv2 = dinv_ref[...]
    pre = (acc_ref[...] + g_ref[...]) * dinv2 + b_ref[...]
    xl = jnp.maximum(pre, 0.01 * pre)
    w = w_ref[...]
    z = jnp.zeros((D, D), jnp.float32)
    wd = jnp.concatenate([jnp.concatenate([w, z], 1),
                          jnp.concatenate([z, w], 1)], 0)
    out_ref[...] = jnp.dot(xl, wd,
                           preferred_element_type=jnp.float32) * dinv2


_mid = pl.pallas_call(
    _mid_body,
    grid=(GRID,),
    in_specs=[
        pl.BlockSpec((RB, 2 * D), lambda i: (i, 0)),
        pl.BlockSpec((RB, 2 * D), lambda i: (i, 0)),
        pl.BlockSpec((RB, 2 * D), lambda i: (i, 0)),
        pl.BlockSpec((1, 2 * D), lambda i: (0, 0)),
        pl.BlockSpec((D, D), lambda i: (0, 0)),
    ],
    out_specs=pl.BlockSpec((RB, 2 * D), lambda i: (i, 0)),
    out_shape=jax.ShapeDtypeStruct((NP2, 2 * D), jnp.float32),
)


def _final_body(acc_ref, g_ref, dinv_ref, b_ref, bi_ref, wfc_ref, bfc_ref,
                out_ref, sums, counts):
    i = pl.program_id(0)

    @pl.when(i == 0)
    def _():
        sums[...] = jnp.zeros_like(sums)
        counts[...] = jnp.zeros_like(counts)

    h2 = jnp.maximum((acc_ref[...] + g_ref[...]) * dinv_ref[...]
                     + b_ref[...], 0.0)
    iot = lax.broadcasted_iota(jnp.int32, (RB, G), 1)
    oh_e = (bi_ref[...][:, 0:1] == iot).astype(jnp.float32)
    oh_o = (bi_ref[...][:, 1:2] == iot).astype(jnp.float32)
    dn = (((0,), (0,)), ((), ()))
    ones = jnp.ones((RB, 1), jnp.float32)
    sums[...] += (
        lax.dot_general(oh_e, h2[:, :D], dn,
                        preferred_element_type=jnp.float32)
        + lax.dot_general(oh_o, h2[:, D:], dn,
                          preferred_element_type=jnp.float32))
    counts[...] += (
        lax.dot_general(oh_e, ones, dn, preferred_element_type=jnp.float32)
        + lax.dot_general(oh_o, ones, dn,
                          preferred_element_type=jnp.float32))

    @pl.when(i == GRID - 1)
    def _():
        mean = sums[...] / jnp.maximum(counts[...], 1.0)
        z = jnp.dot(mean, wfc_ref[...],
                    preferred_element_type=jnp.float32) + bfc_ref[...]
        out_ref[...] = jax.nn.sigmoid(z)


_final = pl.pallas_call(
    _final_body,
    grid=(GRID,),
    in_specs=[
        pl.BlockSpec((RB, 2 * D), lambda i: (i, 0)),
        pl.BlockSpec((RB, 2 * D), lambda i: (i, 0)),
        pl.BlockSpec((RB, 2 * D), lambda i: (i, 0)),
        pl.BlockSpec((1, 2 * D), lambda i: (0, 0)),
        pl.BlockSpec((RB, 2), lambda i: (i, 0)),
        pl.BlockSpec((D, 1), lambda i: (0, 0)),
        pl.BlockSpec((1, 1), lambda i: (0, 0)),
    ],
    out_specs=pl.BlockSpec((G, 1), lambda i: (0, 0)),
    out_shape=jax.ShapeDtypeStruct((G, 1), jnp.float32),
    scratch_shapes=[
        pltpu.VMEM((G, D), jnp.float32),
        pltpu.VMEM((G, 1), jnp.float32),
    ],
)


def kernel(x, edge_index, batch_index, W0, b0, W1, b1, W2, b2, W3, b3,
           Wfc, bfc):
    src = edge_index[0].astype(jnp.int32)
    dst = edge_index[1].astype(jnp.int32)
    pad_e = E_PAD - E
    src_p = jnp.concatenate(
        [src, (jnp.arange(pad_e, dtype=jnp.int32) % 64)])
    dst_p = jnp.concatenate(
        [dst, jnp.full((pad_e,), 1 << 29, jnp.int32)])
    xp = jnp.zeros((NP, DIN), jnp.float32).at[:N].set(x)
    bi = jnp.concatenate(
        [batch_index.astype(jnp.int32),
         jnp.full((NP - N,), G + 7, jnp.int32)]).reshape(NP, 1)
    zeros2 = jnp.zeros((128, D), jnp.float32)

    zeros16 = jnp.zeros((128, 16), jnp.float32)
    deg, plist, cnts = _part(src_p, dst_p)
    x2 = xp.reshape(NP2, 2 * DIN)
    deg2p = deg.reshape(NP2, 2)
    batch2 = bi.reshape(NP2, 2)
    u2, dinv2 = _u(x2, deg2p)
    acc0 = _agg16(plist, cnts, u2.reshape(NP, 16), zeros16)
    b02 = jnp.concatenate([b0, b0]).reshape(1, 2 * D)
    g2 = _mid0(acc0.reshape(NP2, 32), u2, dinv2, b02, W0, W1)
    acc = _agg(plist, cnts, g2.reshape(NP, D), zeros2)
    for bprev, wnext in ((b1, W2), (b2, W3)):
        b2 = jnp.concatenate([bprev, bprev]).reshape(1, 2 * D)
        g2 = _mid(acc.reshape(NP2, 2 * D), g2, dinv2, b2, wnext)
        acc = _agg(plist, cnts, g2.reshape(NP, D), zeros2)
    b32 = jnp.concatenate([b3, b3]).reshape(1, 2 * D)
    return _final(acc.reshape(NP2, 2 * D), g2, dinv2, b32, batch2,
                  Wfc, bfc.reshape(1, 1))
